# Initial kernel scaffold; baseline (speedup 1.0000x reference)
import jax
import jax.numpy as jnp
from jax.experimental import pallas as pl


def kernel(inputs, W, b):
    raise NotImplementedError("write your pallas kernel here")


# speedup vs baseline: 1.8825x; 1.8825x over previous
import functools

import jax
import jax.numpy as jnp
from jax import lax
from jax.experimental import pallas as pl
from jax.experimental.pallas import tpu as pltpu

B = 128
D_MODEL = 1024
VOCAB = 1000
VPAD = 1024
TOP_P = 0.9
NEG = -1e30


def _body(wt_ref, xt_ref, b_ref, g_ref, out_ref):
    logits = jnp.dot(wt_ref[...], xt_ref[...],
                     preferred_element_type=jnp.float32)
    logits = logits + b_ref[...]

    m = jnp.max(logits, axis=0, keepdims=True)
    e = jnp.exp(logits - m)
    s = jnp.sum(e, axis=0, keepdims=True)
    p = e / s

    row = lax.broadcasted_iota(jnp.int32, (VPAD, B), 0)
    idx = row

    k = 2
    while k <= VPAD:
        dir_first = (row & k) == 0
        j = k // 2
        while j >= 1:
            is_lo = (row & j) == 0
            p_dn = pltpu.roll(p, VPAD - j, axis=0)
            p_up = pltpu.roll(p, j, axis=0)
            i_dn = pltpu.roll(idx, VPAD - j, axis=0)
            i_up = pltpu.roll(idx, j, axis=0)
            pp = jnp.where(is_lo, p_dn, p_up)
            pi = jnp.where(is_lo, i_dn, i_up)
            self_first = (p > pp) | ((p == pp) & (idx < pi))
            keep_self = self_first == (is_lo == dir_first)
            p = jnp.where(keep_self, p, pp)
            idx = jnp.where(keep_self, idx, pi)
            j //= 2
        k *= 2

    c = p
    sh = 1
    while sh < VPAD:
        c = c + jnp.where(row >= sh, pltpu.roll(c, sh, axis=0), 0.0)
        sh *= 2

    mask = c <= TOP_P
    logp = jnp.log(p)
    v = jnp.where(mask, logp, NEG) + g_ref[...]

    vmax = jnp.max(jnp.max(v, axis=0, keepdims=True), axis=1, keepdims=True)
    col = lax.broadcasted_iota(jnp.int32, (VPAD, B), 1)
    lin = row * B + col
    cand = jnp.where(v == vmax, lin, jnp.int32(2**30))
    lin_star = jnp.min(jnp.min(cand, axis=0, keepdims=True),
                       axis=1, keepdims=True)
    tok = jnp.sum(jnp.sum(jnp.where(lin == lin_star, idx, 0),
                          axis=0, keepdims=True), axis=1, keepdims=True)
    out_ref[0, 0] = tok[0, 0]


@jax.jit
def kernel(inputs, W, b):
    xt = inputs.T
    wt = jnp.zeros((VPAD, D_MODEL), jnp.float32).at[:VOCAB].set(W.T)
    bp = jnp.full((VPAD, 1), NEG, jnp.float32).at[:VOCAB, 0].set(b)

    g = jax.random.gumbel(jax.random.key(1234), (B * VOCAB,),
                          dtype=jnp.float32)
    gt = jnp.zeros((VPAD, B), jnp.float32).at[:VOCAB].set(
        g.reshape(B, VOCAB).T)

    tok = pl.pallas_call(
        _body,
        out_shape=jax.ShapeDtypeStruct((1, 1), jnp.int32),
        out_specs=pl.BlockSpec(memory_space=pltpu.SMEM),
    )(wt, xt, bp, gt)
    return tok[0, 0]


# speedup vs baseline: 3.0821x; 1.6373x over previous
import jax
import jax.numpy as jnp
from jax import lax
from jax.experimental import pallas as pl
from jax.experimental.pallas import tpu as pltpu

B = 128
D_MODEL = 1024
VOCAB = 1000
VPAD = 1024
TOP_P = 0.9
NEG = -1e30


def _body(wt_ref, xt_ref, b_ref, eg_ref, out_ref):
    logits = jnp.dot(wt_ref[...], xt_ref[...],
                     preferred_element_type=jnp.float32)
    logits = logits + b_ref[...]

    m = jnp.max(logits, axis=0, keepdims=True)
    e = jnp.exp(logits - m)
    s = jnp.sum(e, axis=0, keepdims=True)
    p_orig = e * (1.0 / s)

    row = lax.broadcasted_iota(jnp.int32, (VPAD, B), 0)
    col = lax.broadcasted_iota(jnp.int32, (VPAD, B), 1)

    p = p_orig
    k = 2
    while k <= VPAD:
        dir_first = (row & k) == 0
        j = k // 2
        while j >= 1:
            is_lo = (row & j) == 0
            p_dn = pltpu.roll(p, VPAD - j, axis=0)
            p_up = pltpu.roll(p, j, axis=0)
            pp = jnp.where(is_lo, p_dn, p_up)
            take_max = is_lo == dir_first
            p = jnp.where(take_max, jnp.maximum(p, pp), jnp.minimum(p, pp))
            j //= 2
        k *= 2

    c = p
    sh = 1
    while sh < VPAD:
        c = c + jnp.where(row >= sh, pltpu.roll(c, sh, axis=0), 0.0)
        sh *= 2

    v = jnp.where(c <= TOP_P, p, 0.0) * eg_ref[...]
    vmax = jnp.max(jnp.max(v, axis=0, keepdims=True), axis=1, keepdims=True)
    lin = row * B + col
    cand = jnp.where(v == vmax, lin, jnp.int32(2**30))
    lin_star = jnp.min(jnp.min(cand, axis=0, keepdims=True),
                       axis=1, keepdims=True)
    r_star = lin_star // B
    b_star = lin_star - r_star * B

    p_star = jnp.sum(jnp.sum(jnp.where(lin == lin_star, p, 0.0),
                             axis=0, keepdims=True), axis=1, keepdims=True)

    colmask = col == b_star
    gt = colmask & (p_orig > p_star)
    cnt_gt = jnp.sum(jnp.sum(jnp.where(gt, 1, 0), axis=0, keepdims=True),
                     axis=1, keepdims=True)
    tie_pos = r_star - cnt_gt
    eq = colmask & (p_orig == p_star)
    eq_i = jnp.where(eq, 1, 0)
    ec = eq_i
    sh = 1
    while sh < VPAD:
        ec = ec + jnp.where(row >= sh, pltpu.roll(ec, sh, axis=0), 0)
        sh *= 2
    win = eq & ((ec - eq_i) == tie_pos)
    tok = jnp.sum(jnp.sum(jnp.where(win, row, 0), axis=0, keepdims=True),
                  axis=1, keepdims=True)
    out_ref[0, 0] = tok[0, 0]


@jax.jit
def kernel(inputs, W, b):
    xt = inputs.T
    wt = jnp.zeros((VPAD, D_MODEL), jnp.float32).at[:VOCAB].set(W.T)
    bp = jnp.full((VPAD, 1), NEG, jnp.float32).at[:VOCAB, 0].set(b)

    g = jax.random.gumbel(jax.random.key(1234), (B * VOCAB,),
                          dtype=jnp.float32)
    eg = jnp.zeros((VPAD, B), jnp.float32).at[:VOCAB].set(
        jnp.exp(g).reshape(B, VOCAB).T)

    tok = pl.pallas_call(
        _body,
        out_shape=jax.ShapeDtypeStruct((1, 1), jnp.int32),
        out_specs=pl.BlockSpec(memory_space=pltpu.SMEM),
    )(wt, xt, bp, eg)
    return tok[0, 0]
